# Initial kernel scaffold; baseline (speedup 1.0000x reference)
import jax
import jax.numpy as jnp
from jax.experimental import pallas as pl


def kernel(x, band_weights, x_prev_tokens, W_ext, ln_g, ln_b, W_gate, b_gate, W1, b1, A1, B1, W2, b2, A2, B2):
    raise NotImplementedError("write your pallas kernel here")


# speedup vs baseline: 2.5691x; 2.5691x over previous
import jax
import jax.numpy as jnp
from jax.experimental import pallas as pl
from jax.experimental.pallas import tpu as pltpu

N = 2048
C = 768
F = 64
E = 8
H = 1536
NB = 4
RNK = 8
ALPHA = 16.0
CAP = 1.25
M_FAN = 2
LOSS_COEF = 0.01
GIN = C + F + 2
K = min(max(1, int(N / float(E) * CAP)), N)
SCALE = ALPHA / float(RNK)

_HI = jax.lax.Precision.HIGHEST
_BF = jnp.bfloat16


def _f32_key(v):
    b = jax.lax.bitcast_convert_type(v, jnp.int32)
    m = jax.lax.shift_right_arithmetic(b, 31)
    return b ^ (m & jnp.int32(0x7FFFFFFF))


def _bdot(a, b):
    return jax.lax.dot_general(a.astype(_BF), b.astype(_BF),
                               (((1,), (0,)), ((), ())),
                               preferred_element_type=jnp.float32)


def _bdot_t(a, b):
    return jax.lax.dot_general(a.astype(_BF), b.astype(_BF),
                               (((1,), (1,)), ((), ())),
                               preferred_element_type=jnp.float32)


def _routing_kernel(x_ref, xp_ref, wext_ref, lngx_ref, lnbx_ref, lngz_ref,
                    lnbz_ref, wgx_ref, wgz_ref, gmisc_ref,
                    gT_ref, loss_ref):
    x = x_ref[...]

    ad = jnp.abs(x - xp_ref[...])
    meanad = jnp.sum(ad, axis=1, keepdims=True) / C
    dev = ad - meanad
    sdv = jnp.sqrt(jnp.sum(dev * dev, axis=1, keepdims=True) / (C - 1))
    mu = jnp.log1p(meanad)
    sd = jnp.log1p(sdv)

    Z = _bdot(x, wext_ref[...])

    ssum = (jnp.sum(x, axis=1, keepdims=True)
            + jnp.sum(Z, axis=1, keepdims=True) + mu + sd)
    mean = ssum / GIN
    dx = x - mean
    dz = Z - mean
    dmu = mu - mean
    dsd = sd - mean
    var = (jnp.sum(dx * dx, axis=1, keepdims=True)
           + jnp.sum(dz * dz, axis=1, keepdims=True)
           + dmu * dmu + dsd * dsd) / GIN
    denom = jnp.sqrt(var + 1e-5)

    nx = dx / denom * lngx_ref[...] + lnbx_ref[...]
    nz = dz / denom * lngz_ref[...] + lnbz_ref[...]
    g_mu = gmisc_ref[0:1, 0:1]
    b_mu = gmisc_ref[0:1, 1:2]
    g_sd = gmisc_ref[0:1, 2:3]
    b_sd = gmisc_ref[0:1, 3:4]
    nmu = dmu / denom * g_mu + b_mu
    nsd = dsd / denom * g_sd + b_sd

    logitsT = _bdot_t(wgx_ref[...], nx)
    logitsT = logitsT + _bdot_t(wgz_ref[...], nz)
    onesT = jnp.ones((1, 1), jnp.float32)
    nmuT = jax.lax.dot_general(onesT, nmu.astype(_BF).astype(jnp.float32),
                               (((1,), (1,)), ((), ())), precision=_HI)
    nsdT = jax.lax.dot_general(onesT, nsd.astype(_BF).astype(jnp.float32),
                               (((1,), (1,)), ((), ())), precision=_HI)
    wmu = gmisc_ref[1:2, 0:E].T.astype(_BF).astype(jnp.float32)
    wsd = gmisc_ref[2:3, 0:E].T.astype(_BF).astype(jnp.float32)
    bgate = gmisc_ref[3:4, 0:E].T
    logitsT = logitsT + wmu * nmuT + wsd * nsdT + bgate

    keys = _f32_key(logitsT)
    cnt0 = jnp.sum((keys >= 0).astype(jnp.float32), axis=1, keepdims=True)
    T = jnp.where(cnt0 >= K, jnp.int32(0), jnp.int32(-2147483648))
    T = jnp.broadcast_to(T, (E, 1))
    for b in range(30, -1, -1):
        Tp = T | jnp.int32(1 << b)
        cnt = jnp.sum((keys >= Tp).astype(jnp.float32), axis=1, keepdims=True)
        T = jnp.where(cnt >= K, Tp, T)
    dispatch = keys >= T

    erow = jax.lax.broadcasted_iota(jnp.int32, (E, N), 0)
    covered = jnp.any(dispatch, axis=0, keepdims=True)
    colmax = jnp.max(logitsT, axis=0, keepdims=True)
    best = jnp.min(jnp.where(logitsT == colmax, erow, E), axis=0, keepdims=True)
    dispatch = dispatch | ((~covered) & (erow == best))

    NEG = jnp.float32(-jnp.inf)
    masked = jnp.where(dispatch, logitsT, NEG)
    val1 = jnp.max(masked, axis=0, keepdims=True)
    idx1 = jnp.min(jnp.where(masked == val1, erow, E), axis=0, keepdims=True)
    masked2 = jnp.where(erow == idx1, NEG, masked)
    val2 = jnp.max(masked2, axis=0, keepdims=True)
    idx2 = jnp.min(jnp.where((masked2 == val2) & (erow != idx1), erow, E),
                   axis=0, keepdims=True)
    oh1 = (erow == idx1)
    oh2 = (erow == idx2)
    sel1 = jnp.sum(jnp.where(oh1, logitsT, 0.0), axis=0, keepdims=True)
    sel2 = jnp.sum(jnp.where(oh2, logitsT, 0.0), axis=0, keepdims=True)
    m = jnp.maximum(sel1, sel2)
    e1 = jnp.exp(sel1 - m)
    e2 = jnp.exp(sel2 - m)
    s = e1 + e2
    w1 = e1 / s
    w2 = e2 / s
    gT = jnp.where(oh1, w1, 0.0) + jnp.where(oh2, w2, 0.0)
    gT_ref[...] = gT

    importance = jnp.sum(gT, axis=1, keepdims=True)
    load = jnp.sum((gT > 0.0).astype(jnp.float32), axis=1, keepdims=True)

    def _cv2(v):
        mn = jnp.mean(v)
        vr = jnp.mean((v - mn) * (v - mn))
        return vr / (mn * mn + 1e-10)

    loss_ref[...] = jnp.reshape((_cv2(importance) + _cv2(load)) * LOSS_COEF,
                                (1, 1))


BT = 512
NT = N // BT


def _expert_kernel(gate_ref, bw_ref, x_ref, w1_ref, b1_ref, a1_ref, b1l_ref,
                   w2_ref, b2_ref, a2_ref, b2l_ref, out_ref):
    e = pl.program_id(0)
    t = pl.program_id(1)
    rows = pl.ds(t * BT, BT)
    x = x_ref[rows, :]
    bw = bw_ref[rows, :]
    h = _bdot(x, w1_ref[0]) + b1_ref[0]
    for b in range(NB):
        xa = _bdot(x, a1_ref[0, b])
        lh = _bdot(xa, b1l_ref[0, b])
        h = h + (SCALE * bw[:, b:b + 1]) * lh
    h = jax.nn.gelu(h)
    out = _bdot(h, w2_ref[0]) + b2_ref[0]
    for b in range(NB):
        ha = _bdot(h, a2_ref[0, b])
        lo = _bdot(ha, b2l_ref[0, b])
        out = out + (SCALE * bw[:, b:b + 1]) * lo
    lane = jax.lax.broadcasted_iota(jnp.int32, (BT, E), 1)
    g = jnp.sum(jnp.where(lane == e, gate_ref[rows, :], 0.0), axis=1,
                keepdims=True)
    contrib = out * g

    @pl.when(e == 0)
    def _():
        out_ref[rows, :] = contrib

    @pl.when(e > 0)
    def _():
        out_ref[rows, :] = out_ref[rows, :] + contrib


def kernel(x, band_weights, x_prev_tokens, W_ext, ln_g, ln_b, W_gate, b_gate,
           W1, b1, A1, B1, W2, b2, A2, B2):
    x = x.astype(jnp.float32)
    xp = x_prev_tokens.astype(jnp.float32)
    lngx = ln_g[None, :C]
    lnbx = ln_b[None, :C]
    lngz = ln_g[None, C:C + F]
    lnbz = ln_b[None, C:C + F]
    wgx = W_gate[:, :C]
    wgz = W_gate[:, C:C + F]
    row0 = jnp.concatenate([ln_g[C + F:C + F + 1], ln_b[C + F:C + F + 1],
                            ln_g[C + F + 1:], ln_b[C + F + 1:],
                            jnp.zeros((E - 4,), jnp.float32)])
    gmisc = jnp.stack([row0, W_gate[:, C + F], W_gate[:, C + F + 1], b_gate],
                      axis=0)

    gT, loss = pl.pallas_call(
        _routing_kernel,
        out_shape=(jax.ShapeDtypeStruct((E, N), jnp.float32),
                   jax.ShapeDtypeStruct((1, 1), jnp.float32)),
        in_specs=[
            pl.BlockSpec((N, C), lambda: (0, 0)),
            pl.BlockSpec((N, C), lambda: (0, 0)),
            pl.BlockSpec((C, F), lambda: (0, 0)),
            pl.BlockSpec((1, C), lambda: (0, 0)),
            pl.BlockSpec((1, C), lambda: (0, 0)),
            pl.BlockSpec((1, F), lambda: (0, 0)),
            pl.BlockSpec((1, F), lambda: (0, 0)),
            pl.BlockSpec((E, C), lambda: (0, 0)),
            pl.BlockSpec((E, F), lambda: (0, 0)),
            pl.BlockSpec((4, E), lambda: (0, 0)),
        ],
        out_specs=(pl.BlockSpec((E, N), lambda: (0, 0)),
                   pl.BlockSpec((1, 1), lambda: (0, 0))),
    )(x, xp, W_ext.T, lngx, lnbx, lngz, lnbz, wgx, wgz, gmisc)

    gating = gT.T

    final = pl.pallas_call(
        _expert_kernel,
        grid=(E, NT),
        out_shape=jax.ShapeDtypeStruct((N, C), jnp.float32),
        in_specs=[
            pl.BlockSpec((N, E), lambda e, t: (0, 0)),
            pl.BlockSpec((N, NB), lambda e, t: (0, 0)),
            pl.BlockSpec((N, C), lambda e, t: (0, 0)),
            pl.BlockSpec((1, C, H), lambda e, t: (e, 0, 0)),
            pl.BlockSpec((1, 1, H), lambda e, t: (e, 0, 0)),
            pl.BlockSpec((1, NB, C, RNK), lambda e, t: (e, 0, 0, 0)),
            pl.BlockSpec((1, NB, RNK, H), lambda e, t: (e, 0, 0, 0)),
            pl.BlockSpec((1, H, C), lambda e, t: (e, 0, 0)),
            pl.BlockSpec((1, 1, C), lambda e, t: (e, 0, 0)),
            pl.BlockSpec((1, NB, H, RNK), lambda e, t: (e, 0, 0, 0)),
            pl.BlockSpec((1, NB, RNK, C), lambda e, t: (e, 0, 0, 0)),
        ],
        out_specs=pl.BlockSpec((N, C), lambda e, t: (0, 0)),
        compiler_params=pltpu.CompilerParams(
            dimension_semantics=("arbitrary", "arbitrary")),
    )(gating, band_weights, x, W1, b1.reshape(E, 1, H), A1, B1,
      W2, b2.reshape(E, 1, C), A2, B2)

    return final, loss[0, 0]


# speedup vs baseline: 3.8734x; 1.5077x over previous
import functools

import jax
import jax.numpy as jnp
from jax import lax
from jax.experimental import pallas as pl
from jax.experimental.pallas import tpu as pltpu
from jax.experimental.pallas import tpu_sc as plsc

N = 2048
C = 768
F = 64
E = 8
H = 1536
NB = 4
RNK = 8
ALPHA = 16.0
CAP = 1.25
M_FAN = 2
LOSS_COEF = 0.01
GIN = C + F + 2
K = min(max(1, int(N / float(E) * CAP)), N)
SCALE = ALPHA / float(RNK)

BTS = 128
NBLK = (2 * N + E * (BTS - 1) + BTS - 1) // BTS
NSLOT = NBLK * BTS
NW = 32
SPT = NSLOT // NW
TPT = N // NW
LR = NB * RNK

_HI = jax.lax.Precision.HIGHEST
_BF = jnp.bfloat16


def _f32_key(v):
    b = jax.lax.bitcast_convert_type(v, jnp.int32)
    m = jax.lax.shift_right_arithmetic(b, 31)
    return b ^ (m & jnp.int32(0x7FFFFFFF))


def _bdot(a, b):
    return jax.lax.dot_general(a.astype(_BF), b.astype(_BF),
                               (((1,), (0,)), ((), ())),
                               preferred_element_type=jnp.float32)


def _bdot_t(a, b):
    return jax.lax.dot_general(a.astype(_BF), b.astype(_BF),
                               (((1,), (1,)), ((), ())),
                               preferred_element_type=jnp.float32)


def _routing_kernel(x_ref, xp_ref, wext_ref, lngx_ref, lnbx_ref, lngz_ref,
                    lnbz_ref, wgx_ref, wgz_ref, gmisc_ref, tri_ref,
                    gT_ref, loss_ref, s1_ref, s2_ref, w1_ref, w2_ref,
                    bexp_ref):
    x = x_ref[...]

    ad = jnp.abs(x - xp_ref[...])
    meanad = jnp.sum(ad, axis=1, keepdims=True) / C
    dev = ad - meanad
    sdv = jnp.sqrt(jnp.sum(dev * dev, axis=1, keepdims=True) / (C - 1))
    mu = jnp.log1p(meanad)
    sd = jnp.log1p(sdv)

    Z = _bdot(x, wext_ref[...])

    ssum = (jnp.sum(x, axis=1, keepdims=True)
            + jnp.sum(Z, axis=1, keepdims=True) + mu + sd)
    mean = ssum / GIN
    dx = x - mean
    dz = Z - mean
    dmu = mu - mean
    dsd = sd - mean
    var = (jnp.sum(dx * dx, axis=1, keepdims=True)
           + jnp.sum(dz * dz, axis=1, keepdims=True)
           + dmu * dmu + dsd * dsd) / GIN
    denom = jnp.sqrt(var + 1e-5)

    nx = dx / denom * lngx_ref[...] + lnbx_ref[...]
    nz = dz / denom * lngz_ref[...] + lnbz_ref[...]
    g_mu = gmisc_ref[0:1, 0:1]
    b_mu = gmisc_ref[0:1, 1:2]
    g_sd = gmisc_ref[0:1, 2:3]
    b_sd = gmisc_ref[0:1, 3:4]
    nmu = dmu / denom * g_mu + b_mu
    nsd = dsd / denom * g_sd + b_sd

    logitsT = _bdot_t(wgx_ref[...], nx)
    logitsT = logitsT + _bdot_t(wgz_ref[...], nz)
    onesT = jnp.ones((1, 1), jnp.float32)
    nmuT = jax.lax.dot_general(onesT, nmu.astype(_BF).astype(jnp.float32),
                               (((1,), (1,)), ((), ())), precision=_HI)
    nsdT = jax.lax.dot_general(onesT, nsd.astype(_BF).astype(jnp.float32),
                               (((1,), (1,)), ((), ())), precision=_HI)
    wmu = gmisc_ref[1:2, 0:E].T.astype(_BF).astype(jnp.float32)
    wsd = gmisc_ref[2:3, 0:E].T.astype(_BF).astype(jnp.float32)
    bgate = gmisc_ref[3:4, 0:E].T
    logitsT = logitsT + wmu * nmuT + wsd * nsdT + bgate

    keys = _f32_key(logitsT)
    cnt0 = jnp.sum((keys >= 0).astype(jnp.float32), axis=1, keepdims=True)
    T = jnp.where(cnt0 >= K, jnp.int32(0), jnp.int32(-2147483648))
    T = jnp.broadcast_to(T, (E, 1))
    for b in range(30, -1, -1):
        Tp = T | jnp.int32(1 << b)
        cnt = jnp.sum((keys >= Tp).astype(jnp.float32), axis=1, keepdims=True)
        T = jnp.where(cnt >= K, Tp, T)
    dispatch = keys >= T

    erow = jax.lax.broadcasted_iota(jnp.int32, (E, N), 0)
    covered = jnp.any(dispatch, axis=0, keepdims=True)
    colmax = jnp.max(logitsT, axis=0, keepdims=True)
    best = jnp.min(jnp.where(logitsT == colmax, erow, E), axis=0, keepdims=True)
    dispatch = dispatch | ((~covered) & (erow == best))

    NEG = jnp.float32(-jnp.inf)
    masked = jnp.where(dispatch, logitsT, NEG)
    val1 = jnp.max(masked, axis=0, keepdims=True)
    idx1 = jnp.min(jnp.where(masked == val1, erow, E), axis=0, keepdims=True)
    masked2 = jnp.where(erow == idx1, NEG, masked)
    val2 = jnp.max(masked2, axis=0, keepdims=True)
    idx2 = jnp.min(jnp.where((masked2 == val2) & (erow != idx1), erow, E),
                   axis=0, keepdims=True)
    oh1 = (erow == idx1)
    oh2 = (erow == idx2)
    sel1 = jnp.sum(jnp.where(oh1, logitsT, 0.0), axis=0, keepdims=True)
    sel2 = jnp.sum(jnp.where(oh2, logitsT, 0.0), axis=0, keepdims=True)
    m = jnp.maximum(sel1, sel2)
    e1 = jnp.exp(sel1 - m)
    e2 = jnp.exp(sel2 - m)
    s = e1 + e2
    w1 = e1 / s
    w2 = e2 / s
    gT = jnp.where(oh1, w1, 0.0) + jnp.where(oh2, w2, 0.0)
    gT_ref[...] = gT
    w1_ref[...] = w1
    w2_ref[...] = w2

    importance = jnp.sum(gT, axis=1, keepdims=True)
    load = jnp.sum((gT > 0.0).astype(jnp.float32), axis=1, keepdims=True)

    def _cv2(v):
        mn = jnp.mean(v)
        vr = jnp.mean((v - mn) * (v - mn))
        return vr / (mn * mn + 1e-10)

    loss_ref[...] = jnp.reshape((_cv2(importance) + _cv2(load)) * LOSS_COEF,
                                (1, 1))

    sel = jnp.where(oh1 | oh2, 1.0, 0.0)
    cum = jax.lax.dot_general(sel.astype(_BF), tri_ref[...],
                              (((1,), (0,)), ((), ())),
                              preferred_element_type=jnp.float32)
    counts = cum[:, N - 1:N]
    nb = jnp.floor((counts + (BTS - 1)) * (1.0 / BTS))
    pc = nb * BTS
    ei = jax.lax.broadcasted_iota(jnp.int32, (E, E), 0)
    ej = jax.lax.broadcasted_iota(jnp.int32, (E, E), 1)
    tri8 = jnp.where(ej < ei, 1.0, 0.0)
    base = jax.lax.dot_general(tri8, pc, (((1,), (0,)), ((), ())),
                               precision=_HI)
    pos = cum - 1.0
    slot1 = jnp.sum(jnp.where(oh1, base + pos, 0.0), axis=0, keepdims=True)
    slot2 = jnp.sum(jnp.where(oh2, base + pos, 0.0), axis=0, keepdims=True)
    s1_ref[...] = slot1.astype(jnp.int32)
    s2_ref[...] = slot2.astype(jnp.int32)

    cnb = (base + pc) * (1.0 / BTS)
    jrow = jax.lax.broadcasted_iota(jnp.int32, (1, 128), 1).astype(jnp.float32)
    be = jnp.sum(jnp.where(jrow >= cnb, 1.0, 0.0), axis=0, keepdims=True)
    bexp_ref[...] = jnp.minimum(be, 7.0).astype(jnp.int32)


def _expert_kernel(bexp_ref, xs_ref, bws_ref, gates_ref, w1_ref, b1_ref,
                   a1_ref, b1c_ref, w2_ref, b2_ref, a2_ref, b2c_ref, ys_ref):
    xs = xs_ref[...]
    bws = bws_ref[:, :LR]
    h = _bdot(xs, w1_ref[0]) + b1_ref[0]
    xa = _bdot(xs, a1_ref[0])
    h = h + _bdot(xa * bws, b1c_ref[0])
    h = jax.nn.gelu(h)
    out = _bdot(h, w2_ref[0]) + b2_ref[0]
    ha = _bdot(h, a2_ref[0])
    out = out + _bdot(ha * bws, b2c_ref[0])
    ys_ref[...] = out * gates_ref[...]


def _sc_dispatch(s1_h, s2_h, w1_h, w2_h, x_h, bwr_h,
                 xs_h, bws_h, gates_h,
                 shared_tok, tok_b, slot_b, w_b, gates_b, idx_l, idx_c,
                 rows_b, bwrows_b, sem):
    cid = lax.axis_index("c")
    sid = lax.axis_index("s")
    wid = sid * 2 + cid

    @pl.when(sid == 0)
    def _():
        def zero_body(i, _):
            tok_b[pl.ds(i * 16, 16)] = jnp.zeros((16,), jnp.int32)
            gates_b[pl.ds(i * 16, 16)] = jnp.zeros((16,), jnp.float32)
            return _
        lax.fori_loop(0, NSLOT // 16, zero_body, None)

        pltpu.sync_copy(s1_h, slot_b)
        pltpu.sync_copy(w1_h, w_b)

        def scat(i, _):
            idx = slot_b[pl.ds(i * 16, 16)]
            vals = lax.iota(jnp.int32, 16) + i * 16
            plsc.store_scatter(tok_b, [idx], vals)
            gv = w_b[pl.ds(i * 16, 16)]
            plsc.store_scatter(gates_b, [idx], gv)
            return _
        lax.fori_loop(0, N // 16, scat, None)

        pltpu.sync_copy(s2_h, slot_b)
        pltpu.sync_copy(w2_h, w_b)
        lax.fori_loop(0, N // 16, scat, None)

        pltpu.sync_copy(tok_b, shared_tok)

        @pl.when(cid == 0)
        def _():
            pltpu.sync_copy(gates_b, gates_h)

    plsc.subcore_barrier()
    base = wid * SPT
    pltpu.sync_copy(shared_tok.at[pl.ds(base, SPT)], idx_l)
    pltpu.async_copy(bwr_h.at[idx_l], bwrows_b, sem).wait()
    pltpu.sync_copy(bwrows_b, bws_h.at[pl.ds(base, SPT)])
    for ch in range(2):
        off = ch * (SPT // 2)
        pltpu.sync_copy(shared_tok.at[pl.ds(base + off, SPT // 2)], idx_c)
        pltpu.async_copy(x_h.at[idx_c], rows_b, sem).wait()
        pltpu.sync_copy(rows_b, xs_h.at[pl.ds(base + off, SPT // 2)])


def _sc_combine(ys_h, s1_h, s2_h, fin_h,
                idx1_l, idx2_l, rows1, rows2, sem):
    cid = lax.axis_index("c")
    sid = lax.axis_index("s")
    wid = sid * 2 + cid
    base = wid * TPT
    pltpu.sync_copy(s1_h.at[pl.ds(base, TPT)], idx1_l)
    pltpu.sync_copy(s2_h.at[pl.ds(base, TPT)], idx2_l)
    pltpu.async_copy(ys_h.at[idx1_l], rows1, sem).wait()
    pltpu.async_copy(ys_h.at[idx2_l], rows2, sem).wait()

    def add_row(r, _):
        def add_chunk(j, __):
            a = rows1[r, pl.ds(j * 16, 16)]
            b = rows2[r, pl.ds(j * 16, 16)]
            rows1[r, pl.ds(j * 16, 16)] = a + b
            return __
        lax.fori_loop(0, C // 16, add_chunk, None)
        return _
    lax.fori_loop(0, TPT, add_row, None)
    pltpu.sync_copy(rows1, fin_h.at[pl.ds(base, TPT)])


def kernel(x, band_weights, x_prev_tokens, W_ext, ln_g, ln_b, W_gate, b_gate,
           W1, b1, A1, B1, W2, b2, A2, B2):
    x = x.astype(jnp.float32)
    xp = x_prev_tokens.astype(jnp.float32)
    lngx = ln_g[None, :C]
    lnbx = ln_b[None, :C]
    lngz = ln_g[None, C:C + F]
    lnbz = ln_b[None, C:C + F]
    wgx = W_gate[:, :C]
    wgz = W_gate[:, C:C + F]
    row0 = jnp.concatenate([ln_g[C + F:C + F + 1], ln_b[C + F:C + F + 1],
                            ln_g[C + F + 1:], ln_b[C + F + 1:],
                            jnp.zeros((E - 4,), jnp.float32)])
    gmisc = jnp.stack([row0, W_gate[:, C + F], W_gate[:, C + F + 1], b_gate],
                      axis=0)
    ti = jax.lax.broadcasted_iota(jnp.int32, (N, N), 0)
    tj = jax.lax.broadcasted_iota(jnp.int32, (N, N), 1)
    tri = jnp.where(ti <= tj, 1.0, 0.0).astype(_BF)

    (gT, loss, s1, s2, w1, w2, bexp) = pl.pallas_call(
        _routing_kernel,
        out_shape=(jax.ShapeDtypeStruct((E, N), jnp.float32),
                   jax.ShapeDtypeStruct((1, 1), jnp.float32),
                   jax.ShapeDtypeStruct((1, N), jnp.int32),
                   jax.ShapeDtypeStruct((1, N), jnp.int32),
                   jax.ShapeDtypeStruct((1, N), jnp.float32),
                   jax.ShapeDtypeStruct((1, N), jnp.float32),
                   jax.ShapeDtypeStruct((1, 128), jnp.int32)),
        in_specs=[
            pl.BlockSpec((N, C), lambda: (0, 0)),
            pl.BlockSpec((N, C), lambda: (0, 0)),
            pl.BlockSpec((C, F), lambda: (0, 0)),
            pl.BlockSpec((1, C), lambda: (0, 0)),
            pl.BlockSpec((1, C), lambda: (0, 0)),
            pl.BlockSpec((1, F), lambda: (0, 0)),
            pl.BlockSpec((1, F), lambda: (0, 0)),
            pl.BlockSpec((E, C), lambda: (0, 0)),
            pl.BlockSpec((E, F), lambda: (0, 0)),
            pl.BlockSpec((4, E), lambda: (0, 0)),
            pl.BlockSpec((N, N), lambda: (0, 0)),
        ],
        out_specs=(pl.BlockSpec((E, N), lambda: (0, 0)),
                   pl.BlockSpec((1, 1), lambda: (0, 0)),
                   pl.BlockSpec((1, N), lambda: (0, 0)),
                   pl.BlockSpec((1, N), lambda: (0, 0)),
                   pl.BlockSpec((1, N), lambda: (0, 0)),
                   pl.BlockSpec((1, N), lambda: (0, 0)),
                   pl.BlockSpec((1, 128), lambda: (0, 0))),
    )(x, xp, W_ext.T, lngx, lnbx, lngz, lnbz, wgx, wgz, gmisc, tri)

    s1f = s1.reshape(N)
    s2f = s2.reshape(N)
    bwrep = jnp.repeat(band_weights * SCALE, RNK, axis=1)
    bwrep = jnp.pad(bwrep, ((0, 0), (0, 128 - LR)))

    mesh = plsc.VectorSubcoreMesh(core_axis_name="c", subcore_axis_name="s")
    xs, bws, gates = pl.kernel(
        _sc_dispatch,
        out_type=(jax.ShapeDtypeStruct((NSLOT, C), jnp.float32),
                  jax.ShapeDtypeStruct((NSLOT, 128), jnp.float32),
                  jax.ShapeDtypeStruct((NSLOT,), jnp.float32)),
        mesh=mesh,
        compiler_params=pltpu.CompilerParams(needs_layout_passes=False),
        scratch_types=[
            pltpu.VMEM_SHARED((NSLOT,), jnp.int32),
            pltpu.VMEM((NSLOT,), jnp.int32),
            pltpu.VMEM((N,), jnp.int32),
            pltpu.VMEM((N,), jnp.float32),
            pltpu.VMEM((NSLOT,), jnp.float32),
            pltpu.VMEM((SPT,), jnp.int32),
            pltpu.VMEM((SPT // 2,), jnp.int32),
            pltpu.VMEM((SPT // 2, C), jnp.float32),
            pltpu.VMEM((SPT, 128), jnp.float32),
            pltpu.SemaphoreType.DMA,
        ],
    )(s1f, s2f, w1.reshape(N), w2.reshape(N), x, bwrep)

    grid_spec = pltpu.PrefetchScalarGridSpec(
        num_scalar_prefetch=1,
        grid=(NBLK,),
        in_specs=[
            pl.BlockSpec((BTS, C), lambda i, be: (i, 0)),
            pl.BlockSpec((BTS, 128), lambda i, be: (i, 0)),
            pl.BlockSpec((BTS, 1), lambda i, be: (i, 0)),
            pl.BlockSpec((1, C, H), lambda i, be: (be[i], 0, 0)),
            pl.BlockSpec((1, 1, H), lambda i, be: (be[i], 0, 0)),
            pl.BlockSpec((1, C, LR), lambda i, be: (be[i], 0, 0)),
            pl.BlockSpec((1, LR, H), lambda i, be: (be[i], 0, 0)),
            pl.BlockSpec((1, H, C), lambda i, be: (be[i], 0, 0)),
            pl.BlockSpec((1, 1, C), lambda i, be: (be[i], 0, 0)),
            pl.BlockSpec((1, H, LR), lambda i, be: (be[i], 0, 0)),
            pl.BlockSpec((1, LR, C), lambda i, be: (be[i], 0, 0)),
        ],
        out_specs=pl.BlockSpec((BTS, C), lambda i, be: (i, 0)),
    )
    A1c = A1.transpose(0, 2, 1, 3).reshape(E, C, LR)
    B1c = B1.reshape(E, LR, H)
    A2c = A2.transpose(0, 2, 1, 3).reshape(E, H, LR)
    B2c = B2.reshape(E, LR, C)
    ys = pl.pallas_call(
        _expert_kernel,
        grid_spec=grid_spec,
        out_shape=jax.ShapeDtypeStruct((NSLOT, C), jnp.float32),
        compiler_params=pltpu.CompilerParams(
            dimension_semantics=("arbitrary",)),
    )(bexp.reshape(128)[:NBLK], xs, bws, gates.reshape(NSLOT, 1),
      W1, b1.reshape(E, 1, H), A1c, B1c, W2, b2.reshape(E, 1, C), A2c, B2c)

    final = pl.kernel(
        _sc_combine,
        out_type=jax.ShapeDtypeStruct((N, C), jnp.float32),
        mesh=mesh,
        compiler_params=pltpu.CompilerParams(needs_layout_passes=False),
        scratch_types=[
            pltpu.VMEM((TPT,), jnp.int32),
            pltpu.VMEM((TPT,), jnp.int32),
            pltpu.VMEM((TPT, C), jnp.float32),
            pltpu.VMEM((TPT, C), jnp.float32),
            pltpu.SemaphoreType.DMA,
        ],
    )(ys, s1f, s2f)

    return final, loss[0, 0]


# speedup vs baseline: 4.0405x; 1.0431x over previous
import functools

import jax
import jax.numpy as jnp
from jax import lax
from jax.experimental import pallas as pl
from jax.experimental.pallas import tpu as pltpu
from jax.experimental.pallas import tpu_sc as plsc

N = 2048
C = 768
F = 64
E = 8
H = 1536
NB = 4
RNK = 8
ALPHA = 16.0
CAP = 1.25
M_FAN = 2
LOSS_COEF = 0.01
GIN = C + F + 2
K = min(max(1, int(N / float(E) * CAP)), N)
SCALE = ALPHA / float(RNK)

BTS = 128
NBLK = (2 * N + E * (BTS - 1) + BTS - 1) // BTS
NSLOT = NBLK * BTS
NW = 32
SPT = NSLOT // NW
TPT = N // NW
LR = NB * RNK

_HI = jax.lax.Precision.HIGHEST
_BF = jnp.bfloat16


def _f32_key(v):
    b = jax.lax.bitcast_convert_type(v, jnp.int32)
    m = jax.lax.shift_right_arithmetic(b, 31)
    return b ^ (m & jnp.int32(0x7FFFFFFF))


def _bdot(a, b):
    return jax.lax.dot_general(a.astype(_BF), b.astype(_BF),
                               (((1,), (0,)), ((), ())),
                               preferred_element_type=jnp.float32)


def _bdot_t(a, b):
    return jax.lax.dot_general(a.astype(_BF), b.astype(_BF),
                               (((1,), (1,)), ((), ())),
                               preferred_element_type=jnp.float32)


def _routing_kernel(x_ref, xp_ref, wext_ref, lngx_ref, lnbx_ref, lngz_ref,
                    lnbz_ref, wgx_ref, wgz_ref, gmisc_ref, tri_ref,
                    gT_ref, loss_ref, s1_ref, s2_ref, w1_ref, w2_ref,
                    bexp_ref):
    x = x_ref[...]

    ad = jnp.abs(x - xp_ref[...])
    meanad = jnp.sum(ad, axis=1, keepdims=True) / C
    dev = ad - meanad
    sdv = jnp.sqrt(jnp.sum(dev * dev, axis=1, keepdims=True) / (C - 1))
    mu = jnp.log1p(meanad)
    sd = jnp.log1p(sdv)

    Z = _bdot(x, wext_ref[...])

    ssum = (jnp.sum(x, axis=1, keepdims=True)
            + jnp.sum(Z, axis=1, keepdims=True) + mu + sd)
    mean = ssum / GIN
    dx = x - mean
    dz = Z - mean
    dmu = mu - mean
    dsd = sd - mean
    var = (jnp.sum(dx * dx, axis=1, keepdims=True)
           + jnp.sum(dz * dz, axis=1, keepdims=True)
           + dmu * dmu + dsd * dsd) / GIN
    denom = jnp.sqrt(var + 1e-5)

    nx = dx / denom * lngx_ref[...] + lnbx_ref[...]
    nz = dz / denom * lngz_ref[...] + lnbz_ref[...]
    g_mu = gmisc_ref[0:1, 0:1]
    b_mu = gmisc_ref[0:1, 1:2]
    g_sd = gmisc_ref[0:1, 2:3]
    b_sd = gmisc_ref[0:1, 3:4]
    nmu = dmu / denom * g_mu + b_mu
    nsd = dsd / denom * g_sd + b_sd

    logitsT = _bdot_t(wgx_ref[...], nx)
    logitsT = logitsT + _bdot_t(wgz_ref[...], nz)
    onesT = jnp.ones((1, 1), jnp.float32)
    nmuT = jax.lax.dot_general(onesT, nmu.astype(_BF).astype(jnp.float32),
                               (((1,), (1,)), ((), ())), precision=_HI)
    nsdT = jax.lax.dot_general(onesT, nsd.astype(_BF).astype(jnp.float32),
                               (((1,), (1,)), ((), ())), precision=_HI)
    wmu = gmisc_ref[1:2, 0:E].T.astype(_BF).astype(jnp.float32)
    wsd = gmisc_ref[2:3, 0:E].T.astype(_BF).astype(jnp.float32)
    bgate = gmisc_ref[3:4, 0:E].T
    logitsT = logitsT + wmu * nmuT + wsd * nsdT + bgate

    keys = _f32_key(logitsT)
    cnt0 = jnp.sum((keys >= 0).astype(jnp.float32), axis=1, keepdims=True)
    T = jnp.where(cnt0 >= K, jnp.int32(0), jnp.int32(-2147483648))
    T = jnp.broadcast_to(T, (E, 1))
    for b in range(30, -1, -1):
        Tp = T | jnp.int32(1 << b)
        cnt = jnp.sum((keys >= Tp).astype(jnp.float32), axis=1, keepdims=True)
        T = jnp.where(cnt >= K, Tp, T)
    dispatch = keys >= T

    erow = jax.lax.broadcasted_iota(jnp.int32, (E, N), 0)
    covered = jnp.any(dispatch, axis=0, keepdims=True)
    colmax = jnp.max(logitsT, axis=0, keepdims=True)
    best = jnp.min(jnp.where(logitsT == colmax, erow, E), axis=0, keepdims=True)
    dispatch = dispatch | ((~covered) & (erow == best))

    NEG = jnp.float32(-jnp.inf)
    masked = jnp.where(dispatch, logitsT, NEG)
    val1 = jnp.max(masked, axis=0, keepdims=True)
    idx1 = jnp.min(jnp.where(masked == val1, erow, E), axis=0, keepdims=True)
    masked2 = jnp.where(erow == idx1, NEG, masked)
    val2 = jnp.max(masked2, axis=0, keepdims=True)
    idx2 = jnp.min(jnp.where((masked2 == val2) & (erow != idx1), erow, E),
                   axis=0, keepdims=True)
    oh1 = (erow == idx1)
    oh2 = (erow == idx2)
    sel1 = jnp.sum(jnp.where(oh1, logitsT, 0.0), axis=0, keepdims=True)
    sel2 = jnp.sum(jnp.where(oh2, logitsT, 0.0), axis=0, keepdims=True)
    m = jnp.maximum(sel1, sel2)
    e1 = jnp.exp(sel1 - m)
    e2 = jnp.exp(sel2 - m)
    s = e1 + e2
    w1 = e1 / s
    w2 = e2 / s
    gT = jnp.where(oh1, w1, 0.0) + jnp.where(oh2, w2, 0.0)
    gT_ref[...] = gT
    w1_ref[...] = w1
    w2_ref[...] = w2

    importance = jnp.sum(gT, axis=1, keepdims=True)
    load = jnp.sum((gT > 0.0).astype(jnp.float32), axis=1, keepdims=True)

    def _cv2(v):
        mn = jnp.mean(v)
        vr = jnp.mean((v - mn) * (v - mn))
        return vr / (mn * mn + 1e-10)

    loss_ref[...] = jnp.reshape((_cv2(importance) + _cv2(load)) * LOSS_COEF,
                                (1, 1))

    sel = jnp.where(oh1 | oh2, 1.0, 0.0)
    cum = jax.lax.dot_general(sel.astype(_BF), tri_ref[...],
                              (((1,), (0,)), ((), ())),
                              preferred_element_type=jnp.float32)
    counts = cum[:, N - 1:N]
    nb = jnp.floor((counts + (BTS - 1)) * (1.0 / BTS))
    pc = nb * BTS
    ei = jax.lax.broadcasted_iota(jnp.int32, (E, E), 0)
    ej = jax.lax.broadcasted_iota(jnp.int32, (E, E), 1)
    tri8 = jnp.where(ej < ei, 1.0, 0.0)
    base = jax.lax.dot_general(tri8, pc, (((1,), (0,)), ((), ())),
                               precision=_HI)
    pos = cum - 1.0
    slot1 = jnp.sum(jnp.where(oh1, base + pos, 0.0), axis=0, keepdims=True)
    slot2 = jnp.sum(jnp.where(oh2, base + pos, 0.0), axis=0, keepdims=True)
    s1_ref[...] = slot1.astype(jnp.int32)
    s2_ref[...] = slot2.astype(jnp.int32)

    cnb = (base + pc) * (1.0 / BTS)
    jrow = jax.lax.broadcasted_iota(jnp.int32, (1, 128), 1).astype(jnp.float32)
    be = jnp.sum(jnp.where(jrow >= cnb, 1.0, 0.0), axis=0, keepdims=True)
    bexp_ref[...] = jnp.minimum(be, 7.0).astype(jnp.int32)


def _expert_kernel(bexp_ref, xs_ref, bws_ref, gates_ref, w1_ref, b1_ref,
                   a1_ref, b1c_ref, w2_ref, b2_ref, a2_ref, b2c_ref, ys_ref):
    xs = xs_ref[...]
    bws = bws_ref[:, :LR]
    h = _bdot(xs, w1_ref[0]) + b1_ref[0]
    xa = _bdot(xs, a1_ref[0])
    h = h + _bdot(xa * bws, b1c_ref[0])
    h = jax.nn.gelu(h)
    out = _bdot(h, w2_ref[0]) + b2_ref[0]
    ha = _bdot(h, a2_ref[0])
    out = out + _bdot(ha * bws, b2c_ref[0])
    ys_ref[...] = out * gates_ref[...]


TSL = N // 16
ZSL = NSLOT // 16
XCH = SPT // 4


def _sc_dispatch(s1_h, s2_h, w1_h, w2_h, x_h, bwr_h,
                 xs_h, bws_h, gates_h,
                 shared_tok, zb, tv_b, sl_b, wv_b, idx_l, idx_a, idx_b,
                 rows_a, rows_b, bwrows_b, sem_a, sem_b, sem_c):
    cid = lax.axis_index("c")
    sid = lax.axis_index("s")
    wid = sid * 2 + cid

    tbase = sid * TSL

    def tv_body(i, _):
        tv_b[pl.ds(i * 16, 16)] = lax.iota(jnp.int32, 16) + (tbase + i * 16)
        zb[pl.ds(i * 16, 16)] = jnp.zeros((16,), jnp.int32)
        return _
    lax.fori_loop(0, TSL // 16, tv_body, None)

    def z_body(i, _):
        zb[pl.ds(TSL + i * 16, 16)] = jnp.zeros((16,), jnp.int32)
        return _
    lax.fori_loop(0, (ZSL - TSL) // 16, z_body, None)
    pltpu.sync_copy(zb, shared_tok.at[pl.ds(sid * ZSL, ZSL)])
    plsc.subcore_barrier()

    pltpu.sync_copy(s1_h.at[pl.ds(tbase, TSL)], sl_b)
    pltpu.sync_copy(tv_b, shared_tok.at[sl_b])

    @pl.when(cid == 0)
    def _():
        pltpu.sync_copy(w1_h.at[pl.ds(tbase, TSL)], wv_b)
        pltpu.sync_copy(wv_b, gates_h.at[sl_b])

    pltpu.sync_copy(s2_h.at[pl.ds(tbase, TSL)], sl_b)
    pltpu.sync_copy(tv_b, shared_tok.at[sl_b])

    @pl.when(cid == 0)
    def _():
        pltpu.sync_copy(w2_h.at[pl.ds(tbase, TSL)], wv_b)
        pltpu.sync_copy(wv_b, gates_h.at[sl_b])

    plsc.subcore_barrier()

    base = wid * SPT
    pltpu.sync_copy(shared_tok.at[pl.ds(base, SPT)], idx_l)
    dbw = pltpu.async_copy(bwr_h.at[idx_l], bwrows_b, sem_c)
    pltpu.sync_copy(shared_tok.at[pl.ds(base, XCH)], idx_a)
    d = pltpu.async_copy(x_h.at[idx_a], rows_a, sem_a)
    for ch in range(4):
        cur_rows = rows_a if ch % 2 == 0 else rows_b
        if ch < 3:
            nxt_idx = idx_b if ch % 2 == 0 else idx_a
            nxt_rows = rows_b if ch % 2 == 0 else rows_a
            nxt_sem = sem_b if ch % 2 == 0 else sem_a
            pltpu.sync_copy(
                shared_tok.at[pl.ds(base + (ch + 1) * XCH, XCH)], nxt_idx)
            dn = pltpu.async_copy(x_h.at[nxt_idx], nxt_rows, nxt_sem)
        d.wait()
        pltpu.sync_copy(cur_rows, xs_h.at[pl.ds(base + ch * XCH, XCH)])
        if ch < 3:
            d = dn
    dbw.wait()
    pltpu.sync_copy(bwrows_b, bws_h.at[pl.ds(base, SPT)])


def _sc_combine(ys_h, s1_h, s2_h, fin_h,
                idx1_l, idx2_l, rows1, rows2, sem):
    cid = lax.axis_index("c")
    sid = lax.axis_index("s")
    wid = sid * 2 + cid
    base = wid * TPT
    pltpu.sync_copy(s1_h.at[pl.ds(base, TPT)], idx1_l)
    pltpu.sync_copy(s2_h.at[pl.ds(base, TPT)], idx2_l)
    pltpu.async_copy(ys_h.at[idx1_l], rows1, sem).wait()
    pltpu.async_copy(ys_h.at[idx2_l], rows2, sem).wait()

    def add_row(r, _):
        def add_chunk(j, __):
            a = rows1[r, pl.ds(j * 16, 16)]
            b = rows2[r, pl.ds(j * 16, 16)]
            rows1[r, pl.ds(j * 16, 16)] = a + b
            return __
        lax.fori_loop(0, C // 16, add_chunk, None)
        return _
    lax.fori_loop(0, TPT, add_row, None)
    pltpu.sync_copy(rows1, fin_h.at[pl.ds(base, TPT)])


def kernel(x, band_weights, x_prev_tokens, W_ext, ln_g, ln_b, W_gate, b_gate,
           W1, b1, A1, B1, W2, b2, A2, B2):
    x = x.astype(jnp.float32)
    xp = x_prev_tokens.astype(jnp.float32)
    lngx = ln_g[None, :C]
    lnbx = ln_b[None, :C]
    lngz = ln_g[None, C:C + F]
    lnbz = ln_b[None, C:C + F]
    wgx = W_gate[:, :C]
    wgz = W_gate[:, C:C + F]
    row0 = jnp.concatenate([ln_g[C + F:C + F + 1], ln_b[C + F:C + F + 1],
                            ln_g[C + F + 1:], ln_b[C + F + 1:],
                            jnp.zeros((E - 4,), jnp.float32)])
    gmisc = jnp.stack([row0, W_gate[:, C + F], W_gate[:, C + F + 1], b_gate],
                      axis=0)
    ti = jax.lax.broadcasted_iota(jnp.int32, (N, N), 0)
    tj = jax.lax.broadcasted_iota(jnp.int32, (N, N), 1)
    tri = jnp.where(ti <= tj, 1.0, 0.0).astype(_BF)

    (gT, loss, s1, s2, w1, w2, bexp) = pl.pallas_call(
        _routing_kernel,
        out_shape=(jax.ShapeDtypeStruct((E, N), jnp.float32),
                   jax.ShapeDtypeStruct((1, 1), jnp.float32),
                   jax.ShapeDtypeStruct((1, N), jnp.int32),
                   jax.ShapeDtypeStruct((1, N), jnp.int32),
                   jax.ShapeDtypeStruct((1, N), jnp.float32),
                   jax.ShapeDtypeStruct((1, N), jnp.float32),
                   jax.ShapeDtypeStruct((1, 128), jnp.int32)),
        in_specs=[
            pl.BlockSpec((N, C), lambda: (0, 0)),
            pl.BlockSpec((N, C), lambda: (0, 0)),
            pl.BlockSpec((C, F), lambda: (0, 0)),
            pl.BlockSpec((1, C), lambda: (0, 0)),
            pl.BlockSpec((1, C), lambda: (0, 0)),
            pl.BlockSpec((1, F), lambda: (0, 0)),
            pl.BlockSpec((1, F), lambda: (0, 0)),
            pl.BlockSpec((E, C), lambda: (0, 0)),
            pl.BlockSpec((E, F), lambda: (0, 0)),
            pl.BlockSpec((4, E), lambda: (0, 0)),
            pl.BlockSpec((N, N), lambda: (0, 0)),
        ],
        out_specs=(pl.BlockSpec((E, N), lambda: (0, 0)),
                   pl.BlockSpec((1, 1), lambda: (0, 0)),
                   pl.BlockSpec((1, N), lambda: (0, 0)),
                   pl.BlockSpec((1, N), lambda: (0, 0)),
                   pl.BlockSpec((1, N), lambda: (0, 0)),
                   pl.BlockSpec((1, N), lambda: (0, 0)),
                   pl.BlockSpec((1, 128), lambda: (0, 0))),
    )(x, xp, W_ext.T, lngx, lnbx, lngz, lnbz, wgx, wgz, gmisc, tri)

    s1f = s1.reshape(N)
    s2f = s2.reshape(N)
    bwrep = jnp.repeat(band_weights * SCALE, RNK, axis=1)
    bwrep = jnp.pad(bwrep, ((0, 0), (0, 128 - LR)))

    mesh = plsc.VectorSubcoreMesh(core_axis_name="c", subcore_axis_name="s")
    xs, bws, gates = pl.kernel(
        _sc_dispatch,
        out_type=(jax.ShapeDtypeStruct((NSLOT, C), jnp.float32),
                  jax.ShapeDtypeStruct((NSLOT, 128), jnp.float32),
                  jax.ShapeDtypeStruct((NSLOT,), jnp.float32)),
        mesh=mesh,
        compiler_params=pltpu.CompilerParams(needs_layout_passes=False),
        scratch_types=[
            pltpu.VMEM_SHARED((NSLOT,), jnp.int32),
            pltpu.VMEM((ZSL,), jnp.int32),
            pltpu.VMEM((TSL,), jnp.int32),
            pltpu.VMEM((TSL,), jnp.int32),
            pltpu.VMEM((TSL,), jnp.float32),
            pltpu.VMEM((SPT,), jnp.int32),
            pltpu.VMEM((XCH,), jnp.int32),
            pltpu.VMEM((XCH,), jnp.int32),
            pltpu.VMEM((XCH, C), jnp.float32),
            pltpu.VMEM((XCH, C), jnp.float32),
            pltpu.VMEM((SPT, 128), jnp.float32),
            pltpu.SemaphoreType.DMA,
            pltpu.SemaphoreType.DMA,
            pltpu.SemaphoreType.DMA,
        ],
    )(s1f, s2f, w1.reshape(N), w2.reshape(N), x, bwrep)

    grid_spec = pltpu.PrefetchScalarGridSpec(
        num_scalar_prefetch=1,
        grid=(NBLK,),
        in_specs=[
            pl.BlockSpec((BTS, C), lambda i, be: (i, 0)),
            pl.BlockSpec((BTS, 128), lambda i, be: (i, 0)),
            pl.BlockSpec((BTS, 1), lambda i, be: (i, 0)),
            pl.BlockSpec((1, C, H), lambda i, be: (be[i], 0, 0)),
            pl.BlockSpec((1, 1, H), lambda i, be: (be[i], 0, 0)),
            pl.BlockSpec((1, C, LR), lambda i, be: (be[i], 0, 0)),
            pl.BlockSpec((1, LR, H), lambda i, be: (be[i], 0, 0)),
            pl.BlockSpec((1, H, C), lambda i, be: (be[i], 0, 0)),
            pl.BlockSpec((1, 1, C), lambda i, be: (be[i], 0, 0)),
            pl.BlockSpec((1, H, LR), lambda i, be: (be[i], 0, 0)),
            pl.BlockSpec((1, LR, C), lambda i, be: (be[i], 0, 0)),
        ],
        out_specs=pl.BlockSpec((BTS, C), lambda i, be: (i, 0)),
    )
    A1c = A1.transpose(0, 2, 1, 3).reshape(E, C, LR)
    B1c = B1.reshape(E, LR, H)
    A2c = A2.transpose(0, 2, 1, 3).reshape(E, H, LR)
    B2c = B2.reshape(E, LR, C)
    ys = pl.pallas_call(
        _expert_kernel,
        grid_spec=grid_spec,
        out_shape=jax.ShapeDtypeStruct((NSLOT, C), jnp.float32),
        compiler_params=pltpu.CompilerParams(
            dimension_semantics=("arbitrary",)),
    )(bexp.reshape(128)[:NBLK], xs, bws, gates.reshape(NSLOT, 1),
      W1, b1.reshape(E, 1, H), A1c, B1c, W2, b2.reshape(E, 1, C), A2c, B2c)

    final = pl.kernel(
        _sc_combine,
        out_type=jax.ShapeDtypeStruct((N, C), jnp.float32),
        mesh=mesh,
        compiler_params=pltpu.CompilerParams(needs_layout_passes=False),
        scratch_types=[
            pltpu.VMEM((TPT,), jnp.int32),
            pltpu.VMEM((TPT,), jnp.int32),
            pltpu.VMEM((TPT, C), jnp.float32),
            pltpu.VMEM((TPT, C), jnp.float32),
            pltpu.SemaphoreType.DMA,
        ],
    )(ys, s1f, s2f)

    return final, loss[0, 0]
